# baseline (device time: 101523 ns/iter reference)
import jax
import jax.numpy as jnp
from jax import lax
from jax.experimental import pallas as pl
from jax.experimental.pallas import tpu as pltpu

N_DEV = 16


def _compute_partial(x, Wg, Wu, Wd):
    m, d = x.shape
    h_per = Wg.shape[1]
    n_out = Wd.shape[1]
    nk = 8
    kblk = h_per // nk

    def body(x_ref, wg_ref, wu_ref, wd_ref, out_ref):
        k = pl.program_id(0)
        gate = jnp.dot(x_ref[...], wg_ref[...], preferred_element_type=jnp.float32)
        up = jnp.dot(x_ref[...], wu_ref[...], preferred_element_type=jnp.float32)
        h = gate * (up * jax.nn.sigmoid(up))
        acc = jnp.dot(h, wd_ref[...], preferred_element_type=jnp.float32)

        @pl.when(k == 0)
        def _():
            out_ref[...] = acc

        @pl.when(k != 0)
        def _():
            out_ref[...] += acc

    return pl.pallas_call(
        body,
        grid=(nk,),
        in_specs=[
            pl.BlockSpec((m, d), lambda k: (0, 0)),
            pl.BlockSpec((d, kblk), lambda k: (0, k)),
            pl.BlockSpec((d, kblk), lambda k: (0, k)),
            pl.BlockSpec((kblk, n_out), lambda k: (k, 0)),
        ],
        out_specs=pl.BlockSpec((m, n_out), lambda k: (0, 0)),
        out_shape=jax.ShapeDtypeStruct((m, n_out), jnp.float32),
        compiler_params=pltpu.CompilerParams(
            dimension_semantics=("arbitrary",),
        ),
    )(x, Wg, Wu, Wd)


def _ring_allreduce(partial):
    m, n = partial.shape
    chunk = m // N_DEV
    half = N_DEV // 2
    sub = 2
    r_rows = chunk // sub

    def rows(ref, c):
        return ref.at[pl.ds(c * chunk, chunk), :]

    def subrows(ref, c, h):
        return ref.at[pl.ds(c * chunk + h * r_rows, r_rows), :]

    def body(
        p_ref,
        out_ref,
        rcomm,
        lcomm,
        rs_r_send,
        rs_r_recv,
        rs_l_send,
        rs_l_recv,
        ag_r_send,
        ag_r_recv,
        ag_l_send,
        ag_l_recv,
    ):
        my = lax.axis_index("i")
        left = lax.rem(my + N_DEV - 1, N_DEV)
        right = lax.rem(my + 1, N_DEV)

        def cidx(k):
            return lax.rem(my + k + 2 * N_DEV, N_DEV)

        barrier_sem = pltpu.get_barrier_semaphore()
        for nbr in (left, right):
            pl.semaphore_signal(
                barrier_sem,
                inc=1,
                device_id=(nbr,),
                device_id_type=pl.DeviceIdType.MESH,
            )
        pl.semaphore_wait(barrier_sem, 2)

        out_ref[...] = p_ref[...]

        def rs_r_rdma(s, h):
            return pltpu.make_async_remote_copy(
                src_ref=subrows(out_ref, cidx(half - s), h),
                dst_ref=rcomm.at[s * sub + h],
                send_sem=rs_r_send.at[s * sub + h],
                recv_sem=rs_r_recv.at[s * sub + h],
                device_id=(right,),
                device_id_type=pl.DeviceIdType.MESH,
            )

        def rs_l_rdma(s, h):
            return pltpu.make_async_remote_copy(
                src_ref=subrows(out_ref, cidx(-(half - 1) + s), h),
                dst_ref=lcomm.at[s * sub + h],
                send_sem=rs_l_send.at[s * sub + h],
                recv_sem=rs_l_recv.at[s * sub + h],
                device_id=(left,),
                device_id_type=pl.DeviceIdType.MESH,
            )

        def ag_r_rdma(s, h):
            c = cidx(-s)
            return pltpu.make_async_remote_copy(
                src_ref=subrows(out_ref, c, h),
                dst_ref=subrows(out_ref, c, h),
                send_sem=ag_r_send.at[s * sub + h],
                recv_sem=ag_r_recv.at[s * sub + h],
                device_id=(right,),
                device_id_type=pl.DeviceIdType.MESH,
            )

        def ag_l_rdma(s, h):
            c = cidx(s)
            return pltpu.make_async_remote_copy(
                src_ref=subrows(out_ref, c, h),
                dst_ref=subrows(out_ref, c, h),
                send_sem=ag_l_send.at[s * sub + h],
                recv_sem=ag_l_recv.at[s * sub + h],
                device_id=(left,),
                device_id_type=pl.DeviceIdType.MESH,
            )

        pending = []

        def start(rdma):
            rdma.start()
            pending.append(rdma)

        for h in range(sub):
            start(rs_r_rdma(0, h))
            start(rs_l_rdma(0, h))
        for s in range(half):
            for h in range(sub):
                rs_r_rdma(s, h).wait_recv()
                subrows(out_ref, cidx(half - 1 - s), h)[...] += rcomm[s * sub + h]
                if s < half - 1:
                    start(rs_r_rdma(s + 1, h))
            if s < half - 1:
                for h in range(sub):
                    rs_l_rdma(s, h).wait_recv()
                    subrows(out_ref, cidx(-(half - 2) + s), h)[...] += lcomm[
                        s * sub + h
                    ]
                    if s < half - 2:
                        start(rs_l_rdma(s + 1, h))

        for h in range(sub):
            start(ag_r_rdma(0, h))
            start(ag_l_rdma(0, h))
        for s in range(half):
            for h in range(sub):
                ag_r_rdma(s, h).wait_recv()
                if s < half - 1:
                    start(ag_r_rdma(s + 1, h))
            if s < half - 1:
                for h in range(sub):
                    ag_l_rdma(s, h).wait_recv()
                    if s < half - 2:
                        start(ag_l_rdma(s + 1, h))

        for rdma in pending:
            rdma.wait_send()

    return pl.pallas_call(
        body,
        out_shape=jax.ShapeDtypeStruct((m, n), jnp.float32),
        in_specs=[pl.BlockSpec(memory_space=pltpu.VMEM)],
        out_specs=pl.BlockSpec(memory_space=pltpu.VMEM),
        scratch_shapes=[
            pltpu.VMEM((half * sub, r_rows, n), jnp.float32),
            pltpu.VMEM(((half - 1) * sub, r_rows, n), jnp.float32),
            pltpu.SemaphoreType.DMA((half * sub,)),
            pltpu.SemaphoreType.DMA((half * sub,)),
            pltpu.SemaphoreType.DMA(((half - 1) * sub,)),
            pltpu.SemaphoreType.DMA(((half - 1) * sub,)),
            pltpu.SemaphoreType.DMA((half * sub,)),
            pltpu.SemaphoreType.DMA((half * sub,)),
            pltpu.SemaphoreType.DMA(((half - 1) * sub,)),
            pltpu.SemaphoreType.DMA(((half - 1) * sub,)),
        ],
        compiler_params=pltpu.CompilerParams(collective_id=0),
    )(partial)


def _fused_mlp_allreduce(x, Wg, Wu, Wd):
    m, d = x.shape
    n = Wd.shape[1]
    chunk = m // N_DEV
    half = N_DEV // 2
    sub = 2
    r_rows = chunk // sub

    def subrows(ref, c, h):
        return ref.at[pl.ds(c * chunk + h * r_rows, r_rows), :]

    def body(
        x_ref,
        wg_ref,
        wu_ref,
        wd_ref,
        out_ref,
        rcomm,
        lcomm,
        rs_r_send,
        rs_r_recv,
        rs_l_send,
        rs_l_recv,
        ag_r_send,
        ag_r_recv,
        ag_l_send,
        ag_l_recv,
    ):
        my = lax.axis_index("i")
        left = lax.rem(my + N_DEV - 1, N_DEV)
        right = lax.rem(my + 1, N_DEV)

        def cidx(k):
            return lax.rem(my + k + 2 * N_DEV, N_DEV)

        def compute_chunk(c):
            xs = x_ref[pl.ds(c * chunk, chunk), :]
            gate = jnp.dot(xs, wg_ref[...], preferred_element_type=jnp.float32)
            up = jnp.dot(xs, wu_ref[...], preferred_element_type=jnp.float32)
            h = gate * (up * jax.nn.sigmoid(up))
            out_ref[pl.ds(c * chunk, chunk), :] = jnp.dot(
                h, wd_ref[...], preferred_element_type=jnp.float32
            )

        barrier_sem = pltpu.get_barrier_semaphore()
        for nbr in (left, right):
            pl.semaphore_signal(
                barrier_sem,
                inc=1,
                device_id=(nbr,),
                device_id_type=pl.DeviceIdType.MESH,
            )
        pl.semaphore_wait(barrier_sem, 2)

        def rs_r_rdma(s, h):
            return pltpu.make_async_remote_copy(
                src_ref=subrows(out_ref, cidx(half - s), h),
                dst_ref=rcomm.at[s * sub + h],
                send_sem=rs_r_send.at[s * sub + h],
                recv_sem=rs_r_recv.at[s * sub + h],
                device_id=(right,),
                device_id_type=pl.DeviceIdType.MESH,
            )

        def rs_l_rdma(s, h):
            return pltpu.make_async_remote_copy(
                src_ref=subrows(out_ref, cidx(-(half - 1) + s), h),
                dst_ref=lcomm.at[s * sub + h],
                send_sem=rs_l_send.at[s * sub + h],
                recv_sem=rs_l_recv.at[s * sub + h],
                device_id=(left,),
                device_id_type=pl.DeviceIdType.MESH,
            )

        def ag_r_rdma(s, h):
            c = cidx(-s)
            return pltpu.make_async_remote_copy(
                src_ref=subrows(out_ref, c, h),
                dst_ref=subrows(out_ref, c, h),
                send_sem=ag_r_send.at[s * sub + h],
                recv_sem=ag_r_recv.at[s * sub + h],
                device_id=(right,),
                device_id_type=pl.DeviceIdType.MESH,
            )

        def ag_l_rdma(s, h):
            c = cidx(s)
            return pltpu.make_async_remote_copy(
                src_ref=subrows(out_ref, c, h),
                dst_ref=subrows(out_ref, c, h),
                send_sem=ag_l_send.at[s * sub + h],
                recv_sem=ag_l_recv.at[s * sub + h],
                device_id=(left,),
                device_id_type=pl.DeviceIdType.MESH,
            )

        pending = []

        def start(rdma):
            rdma.start()
            pending.append(rdma)

        compute_chunk(cidx(half))
        for h in range(sub):
            start(rs_r_rdma(0, h))
        compute_chunk(cidx(-(half - 1)))
        for h in range(sub):
            start(rs_l_rdma(0, h))
        compute_chunk(cidx(half - 1))
        compute_chunk(cidx(-(half - 2)))

        for s in range(half):
            if s < half - 2:
                compute_chunk(cidx(half - 2 - s))
                compute_chunk(cidx(-(half - 3) + s))
            for h in range(sub):
                rs_r_rdma(s, h).wait_recv()
                subrows(out_ref, cidx(half - 1 - s), h)[...] += rcomm[s * sub + h]
                if s < half - 1:
                    start(rs_r_rdma(s + 1, h))
            if s < half - 1:
                for h in range(sub):
                    rs_l_rdma(s, h).wait_recv()
                    subrows(out_ref, cidx(-(half - 2) + s), h)[...] += lcomm[
                        s * sub + h
                    ]
                    if s < half - 2:
                        start(rs_l_rdma(s + 1, h))

        for h in range(sub):
            start(ag_r_rdma(0, h))
            start(ag_l_rdma(0, h))
        for s in range(half):
            for h in range(sub):
                ag_r_rdma(s, h).wait_recv()
                if s < half - 1:
                    start(ag_r_rdma(s + 1, h))
            if s < half - 1:
                for h in range(sub):
                    ag_l_rdma(s, h).wait_recv()
                    if s < half - 2:
                        start(ag_l_rdma(s + 1, h))

        for rdma in pending:
            rdma.wait_send()

    return pl.pallas_call(
        body,
        out_shape=jax.ShapeDtypeStruct((m, n), jnp.float32),
        in_specs=[
            pl.BlockSpec(memory_space=pltpu.VMEM),
            pl.BlockSpec(memory_space=pltpu.VMEM),
            pl.BlockSpec(memory_space=pltpu.VMEM),
            pl.BlockSpec(memory_space=pltpu.VMEM),
        ],
        out_specs=pl.BlockSpec(memory_space=pltpu.VMEM),
        scratch_shapes=[
            pltpu.VMEM((half * sub, r_rows, n), jnp.float32),
            pltpu.VMEM(((half - 1) * sub, r_rows, n), jnp.float32),
            pltpu.SemaphoreType.DMA((half * sub,)),
            pltpu.SemaphoreType.DMA((half * sub,)),
            pltpu.SemaphoreType.DMA(((half - 1) * sub,)),
            pltpu.SemaphoreType.DMA(((half - 1) * sub,)),
            pltpu.SemaphoreType.DMA((half * sub,)),
            pltpu.SemaphoreType.DMA((half * sub,)),
            pltpu.SemaphoreType.DMA(((half - 1) * sub,)),
            pltpu.SemaphoreType.DMA(((half - 1) * sub,)),
        ],
        compiler_params=pltpu.CompilerParams(collective_id=0),
    )(x, Wg, Wu, Wd)


def kernel(x, Wg, Wu, Wd):
    import os

    mode = os.environ.get("KERNEL_SPLIT", "")
    if mode == "compute":
        return _compute_partial(x, Wg, Wu, Wd)
    if mode == "ar":
        return _ring_allreduce(x)
    if mode == "unfused":
        partial = _compute_partial(x, Wg, Wu, Wd)
        return _ring_allreduce(partial)
    return _fused_mlp_allreduce(x, Wg, Wu, Wd)


# device time: 78191 ns/iter; 1.2984x vs baseline; 1.2984x over previous
import jax
import jax.numpy as jnp
from jax import lax
from jax.experimental import pallas as pl
from jax.experimental.pallas import tpu as pltpu

N_DEV = 16


def _compute_partial(x, Wg, Wu, Wd):
    m, d = x.shape
    h_per = Wg.shape[1]
    n_out = Wd.shape[1]
    nk = 4
    kblk = h_per // nk

    def body(x_ref, wg_ref, wu_ref, wd_ref, out_ref):
        k = pl.program_id(0)
        gate = jnp.dot(x_ref[...], wg_ref[...], preferred_element_type=jnp.float32)
        up = jnp.dot(x_ref[...], wu_ref[...], preferred_element_type=jnp.float32)
        h = gate * (up * jax.nn.sigmoid(up))
        acc = jnp.dot(h, wd_ref[...], preferred_element_type=jnp.float32)

        @pl.when(k == 0)
        def _():
            out_ref[...] = acc

        @pl.when(k != 0)
        def _():
            out_ref[...] += acc

    return pl.pallas_call(
        body,
        grid=(nk,),
        in_specs=[
            pl.BlockSpec((m, d), lambda k: (0, 0)),
            pl.BlockSpec((d, kblk), lambda k: (0, k)),
            pl.BlockSpec((d, kblk), lambda k: (0, k)),
            pl.BlockSpec((kblk, n_out), lambda k: (k, 0)),
        ],
        out_specs=pl.BlockSpec((m, n_out), lambda k: (0, 0)),
        out_shape=jax.ShapeDtypeStruct((m, n_out), jnp.float32),
        compiler_params=pltpu.CompilerParams(
            dimension_semantics=("arbitrary",),
        ),
    )(x, Wg, Wu, Wd)


def _ring_allreduce(partial):
    m, n = partial.shape
    chunk = m // N_DEV
    half = N_DEV // 2
    sub = 2
    r_rows = chunk // sub

    def rows(ref, c):
        return ref.at[pl.ds(c * chunk, chunk), :]

    def subrows(ref, c, h):
        return ref.at[pl.ds(c * chunk + h * r_rows, r_rows), :]

    def body(
        p_ref,
        out_ref,
        rcomm,
        lcomm,
        rs_r_send,
        rs_r_recv,
        rs_l_send,
        rs_l_recv,
        ag_r_send,
        ag_r_recv,
        ag_l_send,
        ag_l_recv,
    ):
        my = lax.axis_index("i")
        left = lax.rem(my + N_DEV - 1, N_DEV)
        right = lax.rem(my + 1, N_DEV)

        def cidx(k):
            return lax.rem(my + k + 2 * N_DEV, N_DEV)

        barrier_sem = pltpu.get_barrier_semaphore()
        for nbr in (left, right):
            pl.semaphore_signal(
                barrier_sem,
                inc=1,
                device_id=(nbr,),
                device_id_type=pl.DeviceIdType.MESH,
            )
        pl.semaphore_wait(barrier_sem, 2)

        out_ref[...] = p_ref[...]

        def rs_r_rdma(s, h):
            return pltpu.make_async_remote_copy(
                src_ref=subrows(out_ref, cidx(half - s), h),
                dst_ref=rcomm.at[s * sub + h],
                send_sem=rs_r_send.at[s * sub + h],
                recv_sem=rs_r_recv.at[s * sub + h],
                device_id=(right,),
                device_id_type=pl.DeviceIdType.MESH,
            )

        def rs_l_rdma(s, h):
            return pltpu.make_async_remote_copy(
                src_ref=subrows(out_ref, cidx(-(half - 1) + s), h),
                dst_ref=lcomm.at[s * sub + h],
                send_sem=rs_l_send.at[s * sub + h],
                recv_sem=rs_l_recv.at[s * sub + h],
                device_id=(left,),
                device_id_type=pl.DeviceIdType.MESH,
            )

        def ag_r_rdma(s, h):
            c = cidx(-s)
            return pltpu.make_async_remote_copy(
                src_ref=subrows(out_ref, c, h),
                dst_ref=subrows(out_ref, c, h),
                send_sem=ag_r_send.at[s * sub + h],
                recv_sem=ag_r_recv.at[s * sub + h],
                device_id=(right,),
                device_id_type=pl.DeviceIdType.MESH,
            )

        def ag_l_rdma(s, h):
            c = cidx(s)
            return pltpu.make_async_remote_copy(
                src_ref=subrows(out_ref, c, h),
                dst_ref=subrows(out_ref, c, h),
                send_sem=ag_l_send.at[s * sub + h],
                recv_sem=ag_l_recv.at[s * sub + h],
                device_id=(left,),
                device_id_type=pl.DeviceIdType.MESH,
            )

        pending = []

        def start(rdma):
            rdma.start()
            pending.append(rdma)

        for h in range(sub):
            start(rs_r_rdma(0, h))
            start(rs_l_rdma(0, h))
        for s in range(half):
            for h in range(sub):
                rs_r_rdma(s, h).wait_recv()
                subrows(out_ref, cidx(half - 1 - s), h)[...] += rcomm[s * sub + h]
                if s < half - 1:
                    start(rs_r_rdma(s + 1, h))
            if s < half - 1:
                for h in range(sub):
                    rs_l_rdma(s, h).wait_recv()
                    subrows(out_ref, cidx(-(half - 2) + s), h)[...] += lcomm[
                        s * sub + h
                    ]
                    if s < half - 2:
                        start(rs_l_rdma(s + 1, h))

        for h in range(sub):
            start(ag_r_rdma(0, h))
            start(ag_l_rdma(0, h))
        for s in range(half):
            for h in range(sub):
                ag_r_rdma(s, h).wait_recv()
                if s < half - 1:
                    start(ag_r_rdma(s + 1, h))
            if s < half - 1:
                for h in range(sub):
                    ag_l_rdma(s, h).wait_recv()
                    if s < half - 2:
                        start(ag_l_rdma(s + 1, h))

        for rdma in pending:
            rdma.wait_send()

    return pl.pallas_call(
        body,
        out_shape=jax.ShapeDtypeStruct((m, n), jnp.float32),
        in_specs=[pl.BlockSpec(memory_space=pltpu.VMEM)],
        out_specs=pl.BlockSpec(memory_space=pltpu.VMEM),
        scratch_shapes=[
            pltpu.VMEM((half * sub, r_rows, n), jnp.float32),
            pltpu.VMEM(((half - 1) * sub, r_rows, n), jnp.float32),
            pltpu.SemaphoreType.DMA((half * sub,)),
            pltpu.SemaphoreType.DMA((half * sub,)),
            pltpu.SemaphoreType.DMA(((half - 1) * sub,)),
            pltpu.SemaphoreType.DMA(((half - 1) * sub,)),
            pltpu.SemaphoreType.DMA((half * sub,)),
            pltpu.SemaphoreType.DMA((half * sub,)),
            pltpu.SemaphoreType.DMA(((half - 1) * sub,)),
            pltpu.SemaphoreType.DMA(((half - 1) * sub,)),
        ],
        compiler_params=pltpu.CompilerParams(collective_id=0),
    )(partial)


def _fused_mlp_allreduce(x, Wg, Wu, Wd):
    m, d = x.shape
    n = Wd.shape[1]
    chunk = m // N_DEV
    half = N_DEV // 2
    sub = 2
    r_rows = chunk // sub

    def subrows(ref, c, h):
        return ref.at[pl.ds(c * chunk + h * r_rows, r_rows), :]

    def body(
        x_ref,
        wg_ref,
        wu_ref,
        wd_ref,
        out_ref,
        acc,
        rcomm,
        lcomm,
        rs_r_send,
        rs_r_recv,
        rs_l_send,
        rs_l_recv,
        ag_r_send,
        ag_r_recv,
        ag_l_send,
        ag_l_recv,
    ):
        my = lax.axis_index("i")
        left = lax.rem(my + N_DEV - 1, N_DEV)
        right = lax.rem(my + 1, N_DEV)

        def cidx(k):
            return lax.rem(my + k + 2 * N_DEV, N_DEV)

        def compute_pair(k1):
            k2 = (k1 + 1) % N_DEV
            xs = jnp.concatenate(
                [
                    x_ref[pl.ds(cidx(k1) * chunk, chunk), :],
                    x_ref[pl.ds(cidx(k1 + 1) * chunk, chunk), :],
                ],
                axis=0,
            )
            gate = jnp.dot(xs, wg_ref[...], preferred_element_type=jnp.float32)
            up = jnp.dot(xs, wu_ref[...], preferred_element_type=jnp.float32)
            h = gate * (up * jax.nn.sigmoid(up))
            res = jnp.dot(h, wd_ref[...], preferred_element_type=jnp.float32)
            if k2 == k1 + 1:
                acc[k1 * chunk : (k1 + 2) * chunk, :] = res
            else:
                acc[k1 * chunk : (k1 + 1) * chunk, :] = res[:chunk, :]
                acc[0:chunk, :] = res[chunk:, :]

        def compute_single(k):
            xs = x_ref[pl.ds(cidx(k) * chunk, chunk), :]
            gate = jnp.dot(xs, wg_ref[...], preferred_element_type=jnp.float32)
            up = jnp.dot(xs, wu_ref[...], preferred_element_type=jnp.float32)
            h = gate * (up * jax.nn.sigmoid(up))
            acc[k * chunk : (k + 1) * chunk, :] = jnp.dot(
                h, wd_ref[...], preferred_element_type=jnp.float32
            )

        barrier_sem = pltpu.get_barrier_semaphore()
        for nbr in (left, right):
            pl.semaphore_signal(
                barrier_sem,
                inc=1,
                device_id=(nbr,),
                device_id_type=pl.DeviceIdType.MESH,
            )
        pl.semaphore_wait(barrier_sem, 2)

        def sub_s(k, h):
            return acc.at[pl.ds(k * chunk + h * r_rows, r_rows), :]

        def rs_r_rdma(s, h):
            return pltpu.make_async_remote_copy(
                src_ref=sub_s(half - s, h),
                dst_ref=rcomm.at[s * sub + h],
                send_sem=rs_r_send.at[s * sub + h],
                recv_sem=rs_r_recv.at[s * sub + h],
                device_id=(right,),
                device_id_type=pl.DeviceIdType.MESH,
            )

        def rs_l_rdma(s, h):
            return pltpu.make_async_remote_copy(
                src_ref=sub_s(half + 1 + s, h),
                dst_ref=lcomm.at[s * sub + h],
                send_sem=rs_l_send.at[s * sub + h],
                recv_sem=rs_l_recv.at[s * sub + h],
                device_id=(left,),
                device_id_type=pl.DeviceIdType.MESH,
            )

        def ag_r_rdma(s, h):
            c = cidx(-s)
            return pltpu.make_async_remote_copy(
                src_ref=subrows(out_ref, c, h),
                dst_ref=subrows(out_ref, c, h),
                send_sem=ag_r_send.at[s * sub + h],
                recv_sem=ag_r_recv.at[s * sub + h],
                device_id=(right,),
                device_id_type=pl.DeviceIdType.MESH,
            )

        def ag_l_rdma(s, h):
            c = cidx(s)
            return pltpu.make_async_remote_copy(
                src_ref=subrows(out_ref, c, h),
                dst_ref=subrows(out_ref, c, h),
                send_sem=ag_l_send.at[s * sub + h],
                recv_sem=ag_l_recv.at[s * sub + h],
                device_id=(left,),
                device_id_type=pl.DeviceIdType.MESH,
            )

        pending = []

        def start(rdma):
            rdma.start()
            pending.append(rdma)

        compute_single(8)
        for h in range(sub):
            start(rs_r_rdma(0, h))
        compute_single(9)
        for h in range(sub):
            start(rs_l_rdma(0, h))
        compute_pair(6)
        compute_pair(10)

        pair_at = {1: 4, 2: 12, 3: 2, 4: 14, 5: 0}
        for s in range(half):
            if s in pair_at:
                compute_pair(pair_at[s])
            for h in range(sub):
                rs_r_rdma(s, h).wait_recv()
                sub_s(half - 1 - s, h)[...] += rcomm[s * sub + h]
                if s < half - 1:
                    start(rs_r_rdma(s + 1, h))
            if s < half - 1:
                for h in range(sub):
                    rs_l_rdma(s, h).wait_recv()
                    sub_s((half + 2 + s) % N_DEV, h)[...] += lcomm[s * sub + h]
                    if s < half - 2:
                        start(rs_l_rdma(s + 1, h))

        out_ref[pl.ds(cidx(0) * chunk, chunk), :] = acc[0:chunk, :]

        for h in range(sub):
            start(ag_r_rdma(0, h))
            start(ag_l_rdma(0, h))
        for s in range(half):
            for h in range(sub):
                ag_r_rdma(s, h).wait_recv()
                if s < half - 1:
                    start(ag_r_rdma(s + 1, h))
            if s < half - 1:
                for h in range(sub):
                    ag_l_rdma(s, h).wait_recv()
                    if s < half - 2:
                        start(ag_l_rdma(s + 1, h))

        for rdma in pending:
            rdma.wait_send()

    return pl.pallas_call(
        body,
        out_shape=jax.ShapeDtypeStruct((m, n), jnp.float32),
        in_specs=[
            pl.BlockSpec(memory_space=pltpu.VMEM),
            pl.BlockSpec(memory_space=pltpu.VMEM),
            pl.BlockSpec(memory_space=pltpu.VMEM),
            pl.BlockSpec(memory_space=pltpu.VMEM),
        ],
        out_specs=pl.BlockSpec(memory_space=pltpu.VMEM),
        scratch_shapes=[
            pltpu.VMEM((m, n), jnp.float32),
            pltpu.VMEM((half * sub, r_rows, n), jnp.float32),
            pltpu.VMEM(((half - 1) * sub, r_rows, n), jnp.float32),
            pltpu.SemaphoreType.DMA((half * sub,)),
            pltpu.SemaphoreType.DMA((half * sub,)),
            pltpu.SemaphoreType.DMA(((half - 1) * sub,)),
            pltpu.SemaphoreType.DMA(((half - 1) * sub,)),
            pltpu.SemaphoreType.DMA((half * sub,)),
            pltpu.SemaphoreType.DMA((half * sub,)),
            pltpu.SemaphoreType.DMA(((half - 1) * sub,)),
            pltpu.SemaphoreType.DMA(((half - 1) * sub,)),
        ],
        compiler_params=pltpu.CompilerParams(collective_id=0),
    )(x, Wg, Wu, Wd)


def _fused_bf16_allreduce(x, Wg, Wu, Wd):
    m, d = x.shape
    n = Wd.shape[1]
    chunk = m // N_DEV
    half = N_DEV // 2

    def body(
        x_ref,
        wg_ref,
        wu_ref,
        wd_ref,
        out_ref,
        acc,
        rcomm,
        lcomm,
        stage_r,
        stage_l,
        own_stage,
        ag_rarr,
        ag_larr,
        rs_r_send,
        rs_r_recv,
        rs_l_send,
        rs_l_recv,
        ag_r_send,
        ag_r_recv,
        ag_l_send,
        ag_l_recv,
    ):
        my = lax.axis_index("i")
        left = lax.rem(my + N_DEV - 1, N_DEV)
        right = lax.rem(my + 1, N_DEV)

        def cidx(k):
            return lax.rem(my + k + 2 * N_DEV, N_DEV)

        def ablk(k):
            return acc.at[pl.ds(k * chunk, chunk), :]

        def compute_pair(k1):
            k2 = (k1 + 1) % N_DEV
            xs = jnp.concatenate(
                [
                    x_ref[pl.ds(cidx(k1) * chunk, chunk), :],
                    x_ref[pl.ds(cidx(k1 + 1) * chunk, chunk), :],
                ],
                axis=0,
            )
            gate = jnp.dot(xs, wg_ref[...], preferred_element_type=jnp.float32)
            up = jnp.dot(xs, wu_ref[...], preferred_element_type=jnp.float32)
            h = gate * (up * jax.nn.sigmoid(up))
            res = jnp.dot(h, wd_ref[...], preferred_element_type=jnp.float32)
            if k2 == k1 + 1:
                acc[k1 * chunk : (k1 + 2) * chunk, :] = res
            else:
                acc[k1 * chunk : (k1 + 1) * chunk, :] = res[:chunk, :]
                acc[0:chunk, :] = res[chunk:, :]

        def compute_single(k):
            xs = x_ref[pl.ds(cidx(k) * chunk, chunk), :]
            gate = jnp.dot(xs, wg_ref[...], preferred_element_type=jnp.float32)
            up = jnp.dot(xs, wu_ref[...], preferred_element_type=jnp.float32)
            h = gate * (up * jax.nn.sigmoid(up))
            acc[k * chunk : (k + 1) * chunk, :] = jnp.dot(
                h, wd_ref[...], preferred_element_type=jnp.float32
            )

        barrier_sem = pltpu.get_barrier_semaphore()
        for nbr in (left, right):
            pl.semaphore_signal(
                barrier_sem,
                inc=1,
                device_id=(nbr,),
                device_id_type=pl.DeviceIdType.MESH,
            )
        pl.semaphore_wait(barrier_sem, 2)

        def rs_r_rdma(s):
            return pltpu.make_async_remote_copy(
                src_ref=stage_r.at[s],
                dst_ref=rcomm.at[s],
                send_sem=rs_r_send.at[s],
                recv_sem=rs_r_recv.at[s],
                device_id=(right,),
                device_id_type=pl.DeviceIdType.MESH,
            )

        def rs_l_rdma(s):
            return pltpu.make_async_remote_copy(
                src_ref=stage_l.at[s],
                dst_ref=lcomm.at[s],
                send_sem=rs_l_send.at[s],
                recv_sem=rs_l_recv.at[s],
                device_id=(left,),
                device_id_type=pl.DeviceIdType.MESH,
            )

        def ag_r_rdma(s):
            return pltpu.make_async_remote_copy(
                src_ref=own_stage if s == 0 else ag_rarr.at[s - 1],
                dst_ref=ag_rarr.at[s],
                send_sem=ag_r_send.at[s],
                recv_sem=ag_r_recv.at[s],
                device_id=(right,),
                device_id_type=pl.DeviceIdType.MESH,
            )

        def ag_l_rdma(s):
            return pltpu.make_async_remote_copy(
                src_ref=own_stage if s == 0 else ag_larr.at[s - 1],
                dst_ref=ag_larr.at[s],
                send_sem=ag_l_send.at[s],
                recv_sem=ag_l_recv.at[s],
                device_id=(left,),
                device_id_type=pl.DeviceIdType.MESH,
            )

        pending = []

        def start(rdma):
            rdma.start()
            pending.append(rdma)

        compute_single(8)
        stage_r[0] = acc[8 * chunk : 9 * chunk, :].astype(jnp.bfloat16)
        start(rs_r_rdma(0))
        compute_single(9)
        stage_l[0] = acc[9 * chunk : 10 * chunk, :].astype(jnp.bfloat16)
        start(rs_l_rdma(0))
        compute_pair(6)
        compute_pair(10)

        pair_at = {1: 4, 2: 12, 3: 2, 4: 14, 5: 0}
        for s in range(half):
            if s in pair_at:
                compute_pair(pair_at[s])
            rs_r_rdma(s).wait_recv()
            kr = half - 1 - s
            ablk(kr)[...] += rcomm[s].astype(jnp.float32)
            if s < half - 1:
                stage_r[s + 1] = acc[kr * chunk : (kr + 1) * chunk, :].astype(
                    jnp.bfloat16
                )
                start(rs_r_rdma(s + 1))
                rs_l_rdma(s).wait_recv()
                kl = (half + 2 + s) % N_DEV
                ablk(kl)[...] += lcomm[s].astype(jnp.float32)
                if s < half - 2:
                    stage_l[s + 1] = acc[kl * chunk : (kl + 1) * chunk, :].astype(
                        jnp.bfloat16
                    )
                    start(rs_l_rdma(s + 1))

        out_ref[pl.ds(cidx(0) * chunk, chunk), :] = acc[0:chunk, :]
        own_stage[...] = acc[0:chunk, :].astype(jnp.bfloat16)

        start(ag_r_rdma(0))
        start(ag_l_rdma(0))
        for s in range(half):
            ag_r_rdma(s).wait_recv()
            if s < half - 1:
                start(ag_r_rdma(s + 1))
            out_ref[pl.ds(cidx(-1 - s) * chunk, chunk), :] = ag_rarr[s].astype(
                jnp.float32
            )
            if s < half - 1:
                ag_l_rdma(s).wait_recv()
                if s < half - 2:
                    start(ag_l_rdma(s + 1))
                out_ref[pl.ds(cidx(1 + s) * chunk, chunk), :] = ag_larr[s].astype(
                    jnp.float32
                )

        for rdma in pending:
            rdma.wait_send()

    bf = jnp.bfloat16
    return pl.pallas_call(
        body,
        out_shape=jax.ShapeDtypeStruct((m, n), jnp.float32),
        in_specs=[
            pl.BlockSpec(memory_space=pltpu.VMEM),
            pl.BlockSpec(memory_space=pltpu.VMEM),
            pl.BlockSpec(memory_space=pltpu.VMEM),
            pl.BlockSpec(memory_space=pltpu.VMEM),
        ],
        out_specs=pl.BlockSpec(memory_space=pltpu.VMEM),
        scratch_shapes=[
            pltpu.VMEM((m, n), jnp.float32),
            pltpu.VMEM((half, chunk, n), bf),
            pltpu.VMEM((half - 1, chunk, n), bf),
            pltpu.VMEM((half, chunk, n), bf),
            pltpu.VMEM((half - 1, chunk, n), bf),
            pltpu.VMEM((chunk, n), bf),
            pltpu.VMEM((half, chunk, n), bf),
            pltpu.VMEM((half - 1, chunk, n), bf),
            pltpu.SemaphoreType.DMA((half,)),
            pltpu.SemaphoreType.DMA((half,)),
            pltpu.SemaphoreType.DMA((half - 1,)),
            pltpu.SemaphoreType.DMA((half - 1,)),
            pltpu.SemaphoreType.DMA((half,)),
            pltpu.SemaphoreType.DMA((half,)),
            pltpu.SemaphoreType.DMA((half - 1,)),
            pltpu.SemaphoreType.DMA((half - 1,)),
        ],
        compiler_params=pltpu.CompilerParams(collective_id=0),
    )(x, Wg, Wu, Wd)


def kernel(x, Wg, Wu, Wd):
    import os

    mode = os.environ.get("KERNEL_SPLIT", "")
    if mode == "compute":
        return _compute_partial(x, Wg, Wu, Wd)
    if mode == "ar":
        return _ring_allreduce(x)
    if mode == "fused":
        return _fused_mlp_allreduce(x, Wg, Wu, Wd)
    if mode == "unfused":
        partial = _compute_partial(x, Wg, Wu, Wd)
        return _ring_allreduce(partial)
    return _fused_bf16_allreduce(x, Wg, Wu, Wd)
